# baseline (speedup 1.0000x reference)
import jax
import jax.numpy as jnp
from jax.experimental import pallas as pl
from jax.experimental.pallas import tpu as pltpu


def _sep_kernel(na_ref, c3_ref, cflat_ref, out_ref):
    b = pl.program_id(0)
    na = na_ref[b]
    c3 = c3_ref[0]
    rows, lanes = out_ref.shape[1], out_ref.shape[2]
    lane = jax.lax.broadcasted_iota(jnp.int32, (8, lanes), 1)
    sub = jax.lax.broadcasted_iota(jnp.int32, (8, lanes), 0)
    s = (lane % 3 == sub).astype(jnp.float32)[:3]
    a = jnp.dot(c3, s, preferred_element_type=jnp.float32)
    bm = cflat_ref[0]
    j = jax.lax.broadcasted_iota(jnp.int32, (rows, lanes), 0)
    m = jax.lax.broadcasted_iota(jnp.int32, (rows, lanes), 1)
    mask = (j < na) & (m < 3 * na)
    out_ref[0] = jnp.where(mask, a - bm, jnp.float32(0.0))


def kernel(coords, num_atoms):
    bsz, flat = coords.shape
    maxa = flat // 3
    c3 = coords.reshape(bsz, maxa, 3)
    na = num_atoms.astype(jnp.int32)
    out = pl.pallas_call(
        _sep_kernel,
        grid_spec=pltpu.PrefetchScalarGridSpec(
            num_scalar_prefetch=1,
            grid=(bsz,),
            in_specs=[
                pl.BlockSpec((1, maxa, 3), lambda b, na_ref: (b, 0, 0)),
                pl.BlockSpec((1, 1, flat), lambda b, na_ref: (b, 0, 0)),
            ],
            out_specs=pl.BlockSpec((1, maxa, flat), lambda b, na_ref: (b, 0, 0)),
        ),
        out_shape=jax.ShapeDtypeStruct((bsz, maxa, flat), jnp.float32),
    )(na, c3, coords.reshape(bsz, 1, flat))
    return out.reshape(bsz, maxa, maxa, 3)
